# baseline (speedup 1.0000x reference)
import functools

import jax
import jax.numpy as jnp
from jax import lax
from jax.experimental import pallas as pl
from jax.experimental.pallas import tpu as pltpu
from jax.experimental.pallas import tpu_sc as plsc

EMB_DIM = 64
BATCH = 4096
HIST = 200
HIST_PAD = 208
NC = 2
NS = 16
NW = NC * NS
ROWS_PER_W = BATCH // NW
K = 4
GROUPS_PER_W = ROWS_PER_W // K
OUT_PAD = 16
TC_BLK = 512


def _sc_sum_kernel(tokens_hbm, emb_hbm, out_hbm, tok_v, rows_v, out_v, sem):
    wid = lax.axis_index("s") * NC + lax.axis_index("c")
    row0 = wid * ROWS_PER_W

    def group_body(g, carry):
        base = row0 + g * K
        pltpu.sync_copy(tokens_hbm.at[pl.ds(base, K)], tok_v)
        copies = []
        for r in range(K):
            copies.append(pltpu.async_copy(
                emb_hbm.at[tok_v.at[r, pl.ds(0, 128)]],
                rows_v.at[pl.ds(r * HIST_PAD, 128)], sem))
            copies.append(pltpu.async_copy(
                emb_hbm.at[tok_v.at[r, pl.ds(128, 80)]],
                rows_v.at[pl.ds(r * HIST_PAD + 128, 80)], sem))
        for cp in copies:
            cp.wait()

        for r in range(K):
            def sum_body(i, acc):
                rb = r * HIST_PAD + 2 * i
                return (acc[0] + rows_v[rb, pl.ds(0, 16)]
                        + rows_v[rb + 1, pl.ds(0, 16)],
                        acc[1] + rows_v[rb, pl.ds(16, 16)]
                        + rows_v[rb + 1, pl.ds(16, 16)],
                        acc[2] + rows_v[rb, pl.ds(32, 16)]
                        + rows_v[rb + 1, pl.ds(32, 16)],
                        acc[3] + rows_v[rb, pl.ds(48, 16)]
                        + rows_v[rb + 1, pl.ds(48, 16)])

            z = jnp.zeros((16,), jnp.float32)
            acc = lax.fori_loop(0, HIST_PAD // 2, sum_body, (z, z, z, z),
                                unroll=2)
            for c in range(4):
                out_v[r, pl.ds(16 * c, 16)] = acc[c]

        pltpu.sync_copy(out_v, out_hbm.at[pl.ds(base, K)])
        return carry

    lax.fori_loop(0, GROUPS_PER_W, group_body, 0)


def _tc_finish_kernel(tokens_ref, sums_ref, e0_ref, wt_ref, b_ref, out_ref):
    nz = jnp.sum((tokens_ref[...] == 0).astype(jnp.float32), axis=1,
                 keepdims=True)
    s = sums_ref[...] - nz * e0_ref[...]
    inv = 1.0 / (jnp.float32(HIST_PAD) - nz)
    dots = jnp.dot(s, wt_ref[...], preferred_element_type=jnp.float32)
    out_ref[...] = dots * inv + b_ref[...]


@jax.jit
def _run(tokens_p, emb, wt_pad, b_pad, e0):
    mesh = plsc.VectorSubcoreMesh(core_axis_name="c", subcore_axis_name="s",
                                  num_cores=NC, num_subcores=NS)
    sums = pl.kernel(
        _sc_sum_kernel,
        out_type=jax.ShapeDtypeStruct((BATCH, EMB_DIM), jnp.float32),
        mesh=mesh,
        scratch_types=[
            pltpu.VMEM((K, HIST_PAD), jnp.int32),
            pltpu.VMEM((K * HIST_PAD, EMB_DIM), jnp.float32),
            pltpu.VMEM((K, EMB_DIM), jnp.float32),
            pltpu.SemaphoreType.DMA,
        ],
        compiler_params=pltpu.CompilerParams(use_tc_tiling_on_sc=False),
    )(tokens_p, emb)

    out = pl.pallas_call(
        _tc_finish_kernel,
        out_shape=jax.ShapeDtypeStruct((BATCH, OUT_PAD), jnp.float32),
        grid=(BATCH // TC_BLK,),
        in_specs=[
            pl.BlockSpec((TC_BLK, HIST_PAD), lambda i: (i, 0)),
            pl.BlockSpec((TC_BLK, EMB_DIM), lambda i: (i, 0)),
            pl.BlockSpec((1, EMB_DIM), lambda i: (0, 0)),
            pl.BlockSpec((EMB_DIM, OUT_PAD), lambda i: (0, 0)),
            pl.BlockSpec((1, OUT_PAD), lambda i: (0, 0)),
        ],
        out_specs=pl.BlockSpec((TC_BLK, OUT_PAD), lambda i: (i, 0)),
    )(tokens_p, sums, e0, wt_pad, b_pad)
    return out


def kernel(tokens, emb, W, b):
    tokens_p = jnp.pad(tokens.astype(jnp.int32), ((0, 0), (0, HIST_PAD - HIST)))
    wt_pad = jnp.pad(W, ((0, OUT_PAD - W.shape[0]), (0, 0))).T
    b_pad = jnp.pad(b, (0, OUT_PAD - b.shape[0]))[None, :]
    e0 = emb[0][None, :]
    out = _run(tokens_p, emb, wt_pad, b_pad, e0)
    return out[:, :5]
